# baseline (speedup 1.0000x reference)
import jax
import jax.numpy as jnp
from jax.experimental import pallas as pl

_B, _D, _N = 8, 256, 2048
_Q, _K = 8, 1024
_TN = 512


def _rvq_body(x_ref, cb_ref, cbp_ref, cn_ref, out_ref):
    x = x_ref[0]
    r = x
    ni = -jax.lax.broadcasted_iota(jnp.int32, (_K, _TN), 0).astype(jnp.float32)
    for q in range(_Q):
        cb = cb_ref[q]
        scores = jax.lax.dot_general(
            cb, r, (((1,), (0,)), ((), ())),
            precision=jax.lax.Precision.DEFAULT,
            preferred_element_type=jnp.float32,
        ) - cn_ref[q]
        m = jnp.max(scores, axis=0, keepdims=True)
        masked = jnp.where(scores == m, ni, -jnp.inf)
        bestn = jnp.max(masked, axis=0, keepdims=True)
        onehot = (ni == bestn).astype(jnp.bfloat16)
        q3 = jax.lax.dot_general(
            cbp_ref[q], onehot, (((0,), (0,)), ((), ())),
            precision=jax.lax.Precision.DEFAULT,
            preferred_element_type=jnp.float32,
        )
        quant = (q3[0:_D] + q3[_D:2 * _D]) + q3[2 * _D:3 * _D]
        r = r - quant
    out_ref[0] = x - r


@jax.jit
def kernel(x, codebooks):
    cn = 0.5 * jnp.sum(codebooks * codebooks, axis=-1, keepdims=True)
    hi = jax.lax.optimization_barrier(codebooks.astype(jnp.bfloat16))
    r1 = jax.lax.optimization_barrier(codebooks - hi.astype(jnp.float32))
    mid = jax.lax.optimization_barrier(r1.astype(jnp.bfloat16))
    lo = (r1 - mid.astype(jnp.float32)).astype(jnp.bfloat16)
    cbp = jnp.concatenate([hi, mid, lo], axis=-1)
    grid = (_B, _N // _TN)
    return pl.pallas_call(
        _rvq_body,
        grid=grid,
        in_specs=[
            pl.BlockSpec((1, _D, _TN), lambda b, n: (b, 0, n)),
            pl.BlockSpec((_Q, _K, _D), lambda b, n: (0, 0, 0)),
            pl.BlockSpec((_Q, _K, 3 * _D), lambda b, n: (0, 0, 0)),
            pl.BlockSpec((_Q, _K, 1), lambda b, n: (0, 0, 0)),
        ],
        out_specs=pl.BlockSpec((1, _D, _TN), lambda b, n: (b, 0, n)),
        out_shape=jax.ShapeDtypeStruct((_B, _D, _N), jnp.float32),
    )(x, codebooks, cbp, cn)


# speedup vs baseline: 1.0444x; 1.0444x over previous
import jax
import jax.numpy as jnp
from jax.experimental import pallas as pl

_B, _D, _N = 8, 256, 2048
_Q, _K = 8, 1024
_TN = 512


_H = _TN // 2


def _rvq_body(x_ref, cb_ref, cbp_ref, cn_ref, out_ref):
    x = x_ref[0]
    rs = [x[:, :_H], x[:, _H:]]
    ni = -jax.lax.broadcasted_iota(jnp.int32, (_K, _H), 0).astype(jnp.float32)
    for q in range(_Q):
        cb = cb_ref[q]
        cbp = cbp_ref[q]
        cn = cn_ref[q]
        for h in range(2):
            r = rs[h]
            scores = jax.lax.dot_general(
                cb, r, (((1,), (0,)), ((), ())),
                precision=jax.lax.Precision.DEFAULT,
                preferred_element_type=jnp.float32,
            ) - cn
            m = jnp.max(scores, axis=0, keepdims=True)
            masked = jnp.where(scores == m, ni, -jnp.inf)
            bestn = jnp.max(masked, axis=0, keepdims=True)
            onehot = (ni == bestn).astype(jnp.bfloat16)
            q3 = jax.lax.dot_general(
                cbp, onehot, (((0,), (0,)), ((), ())),
                precision=jax.lax.Precision.DEFAULT,
                preferred_element_type=jnp.float32,
            )
            quant = (q3[0:_D] + q3[_D:2 * _D]) + q3[2 * _D:3 * _D]
            rs[h] = r - quant
    out_ref[0] = x - jnp.concatenate(rs, axis=1)


@jax.jit
def kernel(x, codebooks):
    cn = 0.5 * jnp.sum(codebooks * codebooks, axis=-1, keepdims=True)
    hi = jax.lax.optimization_barrier(codebooks.astype(jnp.bfloat16))
    r1 = jax.lax.optimization_barrier(codebooks - hi.astype(jnp.float32))
    mid = jax.lax.optimization_barrier(r1.astype(jnp.bfloat16))
    lo = (r1 - mid.astype(jnp.float32)).astype(jnp.bfloat16)
    cbp = jnp.concatenate([hi, mid, lo], axis=-1)
    grid = (_B, _N // _TN)
    return pl.pallas_call(
        _rvq_body,
        grid=grid,
        in_specs=[
            pl.BlockSpec((1, _D, _TN), lambda b, n: (b, 0, n)),
            pl.BlockSpec((_Q, _K, _D), lambda b, n: (0, 0, 0)),
            pl.BlockSpec((_Q, _K, 3 * _D), lambda b, n: (0, 0, 0)),
            pl.BlockSpec((_Q, _K, 1), lambda b, n: (0, 0, 0)),
        ],
        out_specs=pl.BlockSpec((1, _D, _TN), lambda b, n: (b, 0, n)),
        out_shape=jax.ShapeDtypeStruct((_B, _D, _N), jnp.float32),
    )(x, codebooks, cbp, cn)


# speedup vs baseline: 1.1579x; 1.1087x over previous
import jax
import jax.numpy as jnp
from jax.experimental import pallas as pl

_B, _D, _N = 8, 256, 2048
_Q, _K = 8, 1024
_TN = 1024


_NH = 2
_H = _TN // _NH


def _rvq_body(x_ref, cb_ref, cbp_ref, cn_ref, out_ref):
    x = x_ref[0]
    rs = [x[:, h * _H:(h + 1) * _H] for h in range(_NH)]
    ni = -jax.lax.broadcasted_iota(jnp.int32, (_K, _H), 0).astype(jnp.float32)
    for q in range(_Q):
        cb = cb_ref[q]
        cbp = cbp_ref[q]
        cn = cn_ref[q]
        scores = [
            jax.lax.dot_general(
                cb, rs[h], (((1,), (0,)), ((), ())),
                precision=jax.lax.Precision.DEFAULT,
                preferred_element_type=jnp.float32,
            ) - cn
            for h in range(_NH)
        ]
        onehots = []
        for h in range(_NH):
            m = jnp.max(scores[h], axis=0, keepdims=True)
            masked = jnp.where(scores[h] == m, ni, -jnp.inf)
            bestn = jnp.max(masked, axis=0, keepdims=True)
            onehots.append((ni == bestn).astype(jnp.bfloat16))
        for h in range(_NH):
            q3 = jax.lax.dot_general(
                cbp, onehots[h], (((1,), (0,)), ((), ())),
                precision=jax.lax.Precision.DEFAULT,
                preferred_element_type=jnp.float32,
            )
            quant = (q3[0:_D] + q3[_D:2 * _D]) + q3[2 * _D:3 * _D]
            rs[h] = rs[h] - quant
    out_ref[0] = x - jnp.concatenate(rs, axis=1)


@jax.jit
def kernel(x, codebooks):
    cn = 0.5 * jnp.sum(codebooks * codebooks, axis=-1, keepdims=True)
    hi = jax.lax.optimization_barrier(codebooks.astype(jnp.bfloat16))
    r1 = jax.lax.optimization_barrier(codebooks - hi.astype(jnp.float32))
    mid = jax.lax.optimization_barrier(r1.astype(jnp.bfloat16))
    lo = (r1 - mid.astype(jnp.float32)).astype(jnp.bfloat16)
    cbp = jnp.transpose(jnp.concatenate([hi, mid, lo], axis=-1),
                        (0, 2, 1))
    grid = (_B, _N // _TN)
    return pl.pallas_call(
        _rvq_body,
        grid=grid,
        in_specs=[
            pl.BlockSpec((1, _D, _TN), lambda b, n: (b, 0, n)),
            pl.BlockSpec((_Q, _K, _D), lambda b, n: (0, 0, 0)),
            pl.BlockSpec((_Q, 3 * _D, _K), lambda b, n: (0, 0, 0)),
            pl.BlockSpec((_Q, _K, 1), lambda b, n: (0, 0, 0)),
        ],
        out_specs=pl.BlockSpec((1, _D, _TN), lambda b, n: (b, 0, n)),
        out_shape=jax.ShapeDtypeStruct((_B, _D, _N), jnp.float32),
    )(x, codebooks, cbp, cn)


# speedup vs baseline: 1.1795x; 1.0186x over previous
import jax
import jax.numpy as jnp
from jax.experimental import pallas as pl

_B, _D, _N = 8, 256, 2048
_Q, _K = 8, 1024
_TN = 2048


_NH = 4
_H = _TN // _NH


def _rvq_body(x_ref, cb_ref, cbp_ref, cn_ref, out_ref):
    x = x_ref[0]
    rs = [x[:, h * _H:(h + 1) * _H] for h in range(_NH)]
    for q in range(_Q):
        cb = cb_ref[q]
        cbp = cbp_ref[q]
        cn = cn_ref[q]
        scores = [
            jax.lax.dot_general(
                cb, rs[h], (((1,), (0,)), ((), ())),
                precision=jax.lax.Precision.DEFAULT,
                preferred_element_type=jnp.float32,
            ) - cn
            for h in range(_NH)
        ]
        onehots = []
        for h in range(_NH):
            ni = -jax.lax.broadcasted_iota(
                jnp.int32, (_K, _H), 0).astype(jnp.float32)
            m = jnp.max(scores[h], axis=0, keepdims=True)
            masked = jnp.where(scores[h] == m, ni, -jnp.inf)
            bestn = jnp.max(masked, axis=0, keepdims=True)
            onehots.append((masked == bestn).astype(jnp.bfloat16))
        for h in range(_NH):
            q3 = jax.lax.dot_general(
                cbp, onehots[h], (((1,), (0,)), ((), ())),
                precision=jax.lax.Precision.DEFAULT,
                preferred_element_type=jnp.float32,
            )
            quant = (q3[0:_D] + q3[_D:2 * _D]) + q3[2 * _D:3 * _D]
            rs[h] = rs[h] - quant
    out_ref[0] = x - jnp.concatenate(rs, axis=1)


@jax.jit
def kernel(x, codebooks):
    cn = 0.5 * jnp.sum(codebooks * codebooks, axis=-1, keepdims=True)
    hi = jax.lax.optimization_barrier(codebooks.astype(jnp.bfloat16))
    r1 = jax.lax.optimization_barrier(codebooks - hi.astype(jnp.float32))
    mid = jax.lax.optimization_barrier(r1.astype(jnp.bfloat16))
    lo = (r1 - mid.astype(jnp.float32)).astype(jnp.bfloat16)
    cbp = jnp.transpose(jnp.concatenate([hi, mid, lo], axis=-1),
                        (0, 2, 1))
    grid = (_B, _N // _TN)
    return pl.pallas_call(
        _rvq_body,
        grid=grid,
        in_specs=[
            pl.BlockSpec((1, _D, _TN), lambda b, n: (b, 0, n)),
            pl.BlockSpec((_Q, _K, _D), lambda b, n: (0, 0, 0)),
            pl.BlockSpec((_Q, 3 * _D, _K), lambda b, n: (0, 0, 0)),
            pl.BlockSpec((_Q, _K, 1), lambda b, n: (0, 0, 0)),
        ],
        out_specs=pl.BlockSpec((1, _D, _TN), lambda b, n: (b, 0, n)),
        out_shape=jax.ShapeDtypeStruct((_B, _D, _N), jnp.float32),
    )(x, codebooks, cbp, cn)
